# baseline (speedup 1.0000x reference)
import jax
import jax.numpy as jnp
from jax import lax
from jax.experimental import pallas as pl
from jax.experimental.pallas import tpu as pltpu
from jax.experimental.pallas import tpu_sc as plsc

_VOCAB = 100000
_D = 64
_B, _S, _W = 1024, 20, 50

_NC = 2
_NS = 16
_NW = _NC * _NS
_DOCS_W = _B // _NW
_HS = _S // 2
_NG = _DOCS_W * 2


def _body(idx_hbm, table_hbm, out_hbm, idx_v, rows0, rows1,
          g0s, g1s, w0s, w1s):
    wid = lax.axis_index("s") * _NC + lax.axis_index("c")
    d0 = wid * _DOCS_W
    pltpu.sync_copy(idx_hbm.at[pl.ds(d0, _DOCS_W)], idx_v)

    def fire_g(dl, s0, rows, sem):
        for b in range(_HS):
            pltpu.async_copy(table_hbm.at[idx_v.at[dl, s0 + b]],
                             rows.at[b], sem)

    def drain_g(dl, s0, rows, sem):
        for b in range(_HS):
            pltpu.make_async_copy(table_hbm.at[idx_v.at[dl, s0 + b]],
                                  rows.at[b], sem).wait()

    def fire_w(dl, s0, rows, sem):
        pltpu.async_copy(rows, out_hbm.at[d0 + dl, pl.ds(s0, _HS)], sem)

    def drain_w(dl, s0, rows, sem):
        pltpu.make_async_copy(rows, out_hbm.at[d0 + dl, pl.ds(s0, _HS)],
                              sem).wait()

    fire_g(0, 0, rows0, g0s)
    drain_g(0, 0, rows0, g0s)
    fire_g(0, _HS, rows1, g1s)
    fire_w(0, 0, rows0, w0s)

    def pair(t, carry):
        drain_g(t, _HS, rows1, g1s)
        drain_w(t, 0, rows0, w0s)
        fire_g(t + 1, 0, rows0, g0s)
        fire_w(t, _HS, rows1, w1s)
        drain_g(t + 1, 0, rows0, g0s)
        drain_w(t, _HS, rows1, w1s)
        fire_g(t + 1, _HS, rows1, g1s)
        fire_w(t + 1, 0, rows0, w0s)
        return carry

    lax.fori_loop(0, _DOCS_W - 1, pair, 0)

    dl = _DOCS_W - 1
    drain_g(dl, _HS, rows1, g1s)
    drain_w(dl, 0, rows0, w0s)
    fire_w(dl, _HS, rows1, w1s)
    drain_w(dl, _HS, rows1, w1s)


@jax.jit
def _gather(idx, table):
    mesh = plsc.VectorSubcoreMesh(
        core_axis_name="c", subcore_axis_name="s",
        num_cores=_NC, num_subcores=_NS)
    f = pl.kernel(
        _body,
        out_type=jax.ShapeDtypeStruct((_B, _S, _W, _D), jnp.float32),
        mesh=mesh,
        scratch_types=[
            pltpu.VMEM((_DOCS_W, _S, _W), jnp.int32),
            pltpu.VMEM((_HS, _W, _D), jnp.float32),
            pltpu.VMEM((_HS, _W, _D), jnp.float32),
            pltpu.SemaphoreType.DMA,
            pltpu.SemaphoreType.DMA,
            pltpu.SemaphoreType.DMA,
            pltpu.SemaphoreType.DMA,
        ],
        compiler_params=pltpu.CompilerParams(use_tc_tiling_on_sc=False),
    )
    return f(idx, table)


def kernel(X, pad_vector, embedding_table):
    return _gather(X.astype(jnp.int32), embedding_table)
